# baseline (speedup 1.0000x reference)
import functools

import jax
import jax.numpy as jnp
from jax.experimental import pallas as pl
from jax.experimental.pallas import tpu as pltpu

D_MODEL = 768
D_FF = 3072
E = 8
K = 2
TEMP = 0.7
N = 2048

TM = 512
FB = 768
NT = N // TM
NF = D_FF // FB


def _router_body(x_ref, wr_ref, br_ref, wdense_ref, aux_ref):
    x = x_ref[...]
    logits = jnp.dot(x, wr_ref[...], preferred_element_type=jnp.float32)
    logits = logits + br_ref[...]
    z = logits / TEMP
    z = z - jnp.max(z, axis=1, keepdims=True)
    ez = jnp.exp(z)
    probs = ez / jnp.sum(ez, axis=1, keepdims=True)

    e_ids = jax.lax.broadcasted_iota(jnp.int32, (N, E), 1)
    m1 = jnp.max(probs, axis=1, keepdims=True)
    i1 = jnp.min(jnp.where(probs == m1, e_ids, E), axis=1, keepdims=True)
    oh1 = e_ids == i1
    p2 = jnp.where(oh1, -jnp.inf, probs)
    m2 = jnp.max(p2, axis=1, keepdims=True)
    i2 = jnp.min(jnp.where(p2 == m2, e_ids, E), axis=1, keepdims=True)
    oh2 = e_ids == i2

    denom = m1 + m2 + 1e-6
    w1 = m1 / denom
    w2 = m2 / denom
    wdense = jnp.where(oh1, w1, 0.0) + jnp.where(oh2, w2, 0.0)
    wdense_ref[...] = wdense

    counts = jnp.sum(oh1.astype(jnp.float32) + oh2.astype(jnp.float32),
                     axis=0, keepdims=True)
    f_i = counts / N
    p_i = jnp.mean(probs, axis=0, keepdims=True)
    aux_ref[...] = E * jnp.sum(f_i * p_i, keepdims=True).reshape(1, 1)


def _router(x_flat, Wr, br):
    return pl.pallas_call(
        _router_body,
        out_shape=[
            jax.ShapeDtypeStruct((N, E), jnp.float32),
            jax.ShapeDtypeStruct((1, 1), jnp.float32),
        ],
    )(x_flat, Wr, br.reshape(1, E))


def _ffn_body(x_ref, w1_ref, b1_ref, w2_ref, b2_ref, wd_ref, out_ref):
    e = pl.program_id(1)
    fb = pl.program_id(2)

    @pl.when((e == 0) & (fb == 0))
    def _():
        out_ref[...] = jnp.zeros_like(out_ref)

    x = x_ref[...]
    h = jnp.dot(x, w1_ref[0], preferred_element_type=jnp.float32)
    h = h + b1_ref[0]
    h = h * (1.0 / (1.0 + jnp.exp(-h)))
    contrib = jnp.dot(h, w2_ref[0], preferred_element_type=jnp.float32)

    e_ids = jax.lax.broadcasted_iota(jnp.int32, (TM, E), 1)
    w_row = jnp.sum(jnp.where(e_ids == e, wd_ref[...], 0.0),
                    axis=1, keepdims=True)

    @pl.when(fb == 0)
    def _():
        out_ref[...] += w_row * (contrib + b2_ref[0])

    @pl.when(fb != 0)
    def _():
        out_ref[...] += w_row * contrib


def _ffn(x_flat, W1, b1, W2, b2, wdense):
    return pl.pallas_call(
        _ffn_body,
        grid=(NT, E, NF),
        in_specs=[
            pl.BlockSpec((TM, D_MODEL), lambda t, e, f: (t, 0)),
            pl.BlockSpec((1, D_MODEL, FB), lambda t, e, f: (e, 0, f)),
            pl.BlockSpec((1, 1, FB), lambda t, e, f: (e, 0, f)),
            pl.BlockSpec((1, FB, D_MODEL), lambda t, e, f: (e, f, 0)),
            pl.BlockSpec((1, 1, D_MODEL), lambda t, e, f: (e, 0, 0)),
            pl.BlockSpec((TM, E), lambda t, e, f: (t, 0)),
        ],
        out_specs=pl.BlockSpec((TM, D_MODEL), lambda t, e, f: (t, 0)),
        out_shape=jax.ShapeDtypeStruct((N, D_MODEL), jnp.float32),
    )(x_flat, W1, b1.reshape(E, 1, D_FF), W2, b2.reshape(E, 1, D_MODEL),
      wdense)


@jax.jit
def kernel(x, Wr, br, W1, b1, W2, b2):
    B, L, D = x.shape
    x_flat = x.reshape(-1, D)
    wdense, aux = _router(x_flat, Wr, br)
    out = _ffn(x_flat, W1, b1, W2, b2, wdense)
    return out.reshape(B, L, D), aux.reshape(1)


# speedup vs baseline: 1.3205x; 1.3205x over previous
import functools

import jax
import jax.numpy as jnp
from jax import lax
from jax.experimental import pallas as pl
from jax.experimental.pallas import tpu as pltpu
from jax.experimental.pallas import tpu_sc as plsc

D_MODEL = 768
D_FF = 3072
E = 8
TEMP = 0.7
N = 2048
NA = 2 * N

TM = 256
NB = NA // TM
T_TILES = NB + E - 1
TPAD = 24
FB = 1536
NF = D_FF // FB

NC, NS = 2, 16
NW = NC * NS
GPW = NA // NW
CPW = N // NW


def _router_body(x_ref, wr_ref, br_ref,
                 posf_ref, wk_ref, posi_ref, cnts_ref, aux_ref):
    x = x_ref[...]
    logits = jnp.dot(x, wr_ref[...], preferred_element_type=jnp.float32)
    logits = logits + br_ref[...]
    z = logits / TEMP
    z = z - jnp.max(z, axis=1, keepdims=True)
    ez = jnp.exp(z)
    probs = ez / jnp.sum(ez, axis=1, keepdims=True)

    e_ids = lax.broadcasted_iota(jnp.int32, (N, E), 1)
    m1 = jnp.max(probs, axis=1, keepdims=True)
    i1 = jnp.min(jnp.where(probs == m1, e_ids, E), axis=1, keepdims=True)
    oh1 = (e_ids == i1).astype(jnp.float32)
    p2 = jnp.where(oh1 > 0, -jnp.inf, probs)
    m2 = jnp.max(p2, axis=1, keepdims=True)
    i2 = jnp.min(jnp.where(p2 == m2, e_ids, E), axis=1, keepdims=True)
    oh2 = (e_ids == i2).astype(jnp.float32)

    denom = m1 + m2 + 1e-6
    w1n = m1 / denom
    w2n = m2 / denom

    c_pair = oh1 + oh2
    t_row = lax.broadcasted_iota(jnp.int32, (N, N), 0)
    t_col = lax.broadcasted_iota(jnp.int32, (N, N), 1)
    tri = (t_col <= t_row).astype(jnp.float32)
    cum = jnp.dot(tri, c_pair, preferred_element_type=jnp.float32)

    counts = cum[N - 1:N, :]
    offs = jnp.zeros_like(counts)
    for k in range(1, E):
        offs = offs + jnp.concatenate(
            [jnp.zeros((1, k), jnp.float32), counts[:, :E - k]], axis=1)

    slot = offs + cum - 1.0
    pos0 = jnp.sum(oh1 * slot, axis=1, keepdims=True)
    pos1 = jnp.sum(oh2 * slot, axis=1, keepdims=True)

    posf = jnp.concatenate([pos0, pos1], axis=1)
    posf_ref[...] = posf
    wk_ref[...] = jnp.concatenate([w1n, w2n], axis=1)
    posi_ref[...] = posf.astype(jnp.int32)
    cnts_ref[...] = jnp.concatenate([counts, offs], axis=0).astype(jnp.int32)

    f_i = counts / N
    p_i = jnp.mean(probs, axis=0, keepdims=True)
    aux_ref[...] = E * jnp.sum(f_i * p_i, keepdims=True).reshape(1, 1)


def _router(x_flat, Wr, br):
    return pl.pallas_call(
        _router_body,
        out_shape=[
            jax.ShapeDtypeStruct((N, 2), jnp.float32),
            jax.ShapeDtypeStruct((N, 2), jnp.float32),
            jax.ShapeDtypeStruct((N, 2), jnp.int32),
            jax.ShapeDtypeStruct((2, E), jnp.int32),
            jax.ShapeDtypeStruct((1, 1), jnp.float32),
        ],
    )(x_flat, Wr, br.reshape(1, E))


_INV_B = 512


def _inv_body(posf_ref, wk_ref, tok_ref, ws_ref):
    b = pl.program_id(0)
    slot = (lax.broadcasted_iota(jnp.int32, (1, _INV_B), 1)
            + _INV_B * b).astype(jnp.float32)
    pos0 = posf_ref[:, 0:1]
    pos1 = posf_ref[:, 1:2]
    m0 = (pos0 == slot).astype(jnp.float32)
    m1 = (pos1 == slot).astype(jnp.float32)
    t_ids = lax.broadcasted_iota(jnp.int32, (N, 1), 0).astype(jnp.float32)
    tok = jnp.sum((m0 + m1) * t_ids, axis=0, keepdims=True)
    w = jnp.sum(m0 * wk_ref[:, 0:1] + m1 * wk_ref[:, 1:2],
                axis=0, keepdims=True)
    tok_ref[0] = tok.astype(jnp.int32)
    ws_ref[0] = w


def _invert(posf, wk):
    nblk = NA // _INV_B
    return pl.pallas_call(
        _inv_body,
        grid=(nblk,),
        in_specs=[
            pl.BlockSpec((N, 2), lambda b: (0, 0)),
            pl.BlockSpec((N, 2), lambda b: (0, 0)),
        ],
        out_specs=[
            pl.BlockSpec((1, 1, _INV_B), lambda b: (b, 0, 0)),
            pl.BlockSpec((1, 1, _INV_B), lambda b: (b, 0, 0)),
        ],
        out_shape=[
            jax.ShapeDtypeStruct((nblk, 1, _INV_B), jnp.int32),
            jax.ShapeDtypeStruct((nblk, 1, _INV_B), jnp.float32),
        ],
    )(posf, wk)


@functools.lru_cache(maxsize=None)
def _sc_mesh():
    return plsc.VectorSubcoreMesh(
        core_axis_name="c", subcore_axis_name="s",
        num_cores=NC, num_subcores=NS)


def _sc_gather(tok_sorted, x_flat):
    @functools.partial(
        pl.kernel,
        out_type=jax.ShapeDtypeStruct((NA, D_MODEL), jnp.float32),
        mesh=_sc_mesh(),
        scratch_types=[
            pltpu.VMEM((GPW,), jnp.int32),
            pltpu.VMEM((GPW, D_MODEL), jnp.float32),
            pltpu.SemaphoreType.DMA,
        ],
    )
    def body(tok_hbm, x_hbm, out_hbm, idx_v, rows_v, sem):
        wid = lax.axis_index("s") * NC + lax.axis_index("c")
        base = wid * GPW
        pltpu.sync_copy(tok_hbm.at[pl.ds(base, GPW)], idx_v)
        pltpu.async_copy(x_hbm.at[idx_v], rows_v, sem).wait()
        pltpu.sync_copy(rows_v, out_hbm.at[pl.ds(base, GPW)])

    return body(tok_sorted, x_flat)


def _ffn_body(meta_ref, x_ref, w1_ref, b1_ref, w2_ref, b2_ref, ws_ref,
              out_ref):
    t = pl.program_id(0)
    f = pl.program_id(1)
    r = meta_ref[1, t]
    r_prev = meta_ref[1, jnp.maximum(t - 1, 0)]
    first = (f == 0) & ((t == 0) | (r != r_prev))
    lo = meta_ref[2, t]
    hi = meta_ref[3, t]

    rows = lax.broadcasted_iota(jnp.int32, (TM, 1), 0)
    valid = (rows >= lo) & (rows < hi)

    x = x_ref[...]
    h = jnp.dot(x, w1_ref[0], preferred_element_type=jnp.float32)
    h = h + b1_ref[0]
    h = h * (1.0 / (1.0 + jnp.exp(-h)))
    contrib = jnp.dot(h, w2_ref[0], preferred_element_type=jnp.float32)
    contrib = contrib + jnp.where(f == 0, 1.0, 0.0) * b2_ref[0]
    update = jnp.where(valid, ws_ref[...] * contrib, 0.0)

    @pl.when(first)
    def _():
        out_ref[...] = update

    @pl.when(jnp.logical_not(first))
    def _():
        out_ref[...] += update


def _ffn_grouped(meta, x_sorted, W1, b1, W2, b2, w_sorted):
    grid_spec = pltpu.PrefetchScalarGridSpec(
        num_scalar_prefetch=1,
        grid=(TPAD, NF),
        in_specs=[
            pl.BlockSpec((TM, D_MODEL), lambda t, f, m: (m[1, t], 0)),
            pl.BlockSpec((1, D_MODEL, FB), lambda t, f, m: (m[0, t], 0, f)),
            pl.BlockSpec((1, 1, FB), lambda t, f, m: (m[0, t], 0, f)),
            pl.BlockSpec((1, FB, D_MODEL), lambda t, f, m: (m[0, t], f, 0)),
            pl.BlockSpec((1, 1, D_MODEL), lambda t, f, m: (m[0, t], 0, 0)),
            pl.BlockSpec((TM, 1), lambda t, f, m: (m[1, t], 0)),
        ],
        out_specs=pl.BlockSpec((TM, D_MODEL), lambda t, f, m: (m[1, t], 0)),
    )
    return pl.pallas_call(
        _ffn_body,
        grid_spec=grid_spec,
        out_shape=jax.ShapeDtypeStruct((NA, D_MODEL), jnp.float32),
    )(meta, x_sorted, W1, b1.reshape(E, 1, D_FF), W2,
      b2.reshape(E, 1, D_MODEL), w_sorted)


def _sc_combine(i0, i1, out_sorted):
    @functools.partial(
        pl.kernel,
        out_type=jax.ShapeDtypeStruct((N, D_MODEL), jnp.float32),
        mesh=_sc_mesh(),
        scratch_types=[
            pltpu.VMEM((CPW,), jnp.int32),
            pltpu.VMEM((CPW,), jnp.int32),
            pltpu.VMEM((CPW, D_MODEL), jnp.float32),
            pltpu.VMEM((CPW, D_MODEL), jnp.float32),
            pltpu.SemaphoreType.DMA,
        ],
    )
    def body(i0_hbm, i1_hbm, os_hbm, out_hbm, idx0, idx1, r0, r1, sem):
        wid = lax.axis_index("s") * NC + lax.axis_index("c")
        base = wid * CPW
        pltpu.sync_copy(i0_hbm.at[pl.ds(base, CPW)], idx0)
        pltpu.sync_copy(i1_hbm.at[pl.ds(base, CPW)], idx1)
        pltpu.async_copy(os_hbm.at[idx0], r0, sem).wait()
        pltpu.async_copy(os_hbm.at[idx1], r1, sem).wait()

        def add_row(i, _):
            for j in range(D_MODEL // 16):
                sl = pl.ds(j * 16, 16)
                r0[i, sl] = r0[i, sl] + r1[i, sl]
            return 0

        lax.fori_loop(0, CPW, add_row, 0)
        pltpu.sync_copy(r0, out_hbm.at[pl.ds(base, CPW)])

    return body(i0, i1, out_sorted)


def _tile_meta(counts, offs):
    starts = offs
    ends = offs + counts
    r_ids = jnp.arange(NB, dtype=jnp.int32)[:, None]
    flag = ((starts[None, :] < (r_ids + 1) * TM)
            & (ends[None, :] > r_ids * TM)
            & (counts[None, :] > 0))
    tidx = jnp.cumsum(flag.reshape(-1).astype(jnp.int32)) - 1
    tgt = jnp.where(flag.reshape(-1), tidx, TPAD)
    lo = jnp.clip(jnp.broadcast_to(starts[None, :], (NB, E)) - r_ids * TM,
                  0, TM).reshape(-1)
    hi = jnp.clip(jnp.broadcast_to(ends[None, :], (NB, E)) - r_ids * TM,
                  0, TM).reshape(-1)
    e_flat = jnp.broadcast_to(jnp.arange(E, dtype=jnp.int32)[None, :],
                              (NB, E)).reshape(-1)
    r_flat = jnp.broadcast_to(r_ids, (NB, E)).reshape(-1)

    e_t = jnp.zeros((TPAD,), jnp.int32).at[tgt].set(e_flat, mode="drop")
    r_t = jnp.full((TPAD,), NB - 1, jnp.int32).at[tgt].set(r_flat,
                                                           mode="drop")
    lo_t = jnp.zeros((TPAD,), jnp.int32).at[tgt].set(lo, mode="drop")
    hi_t = jnp.zeros((TPAD,), jnp.int32).at[tgt].set(hi, mode="drop")
    return jnp.stack([e_t, r_t, lo_t, hi_t])


@jax.jit
def kernel(x, Wr, br, W1, b1, W2, b2):
    B, L, D = x.shape
    x_flat = x.reshape(-1, D)
    posf, wk, posi, cnts, aux = _router(x_flat, Wr, br)
    tok3, ws3 = _invert(posf, wk)
    tok_sorted = tok3.reshape(-1)
    w_sorted = ws3.reshape(-1, 1)
    meta = _tile_meta(cnts[0], cnts[1])
    x_sorted = _sc_gather(tok_sorted, x_flat)
    out_sorted = _ffn_grouped(meta, x_sorted, W1, b1, W2, b2, w_sorted)
    i0 = posi[:, 0]
    i1 = posi[:, 1]
    out = _sc_combine(i0, i1, out_sorted)
    return out.reshape(B, L, D), aux.reshape(1)


# speedup vs baseline: 1.5998x; 1.2115x over previous
import functools

import jax
import jax.numpy as jnp
from jax import lax
from jax.experimental import pallas as pl
from jax.experimental.pallas import tpu as pltpu
from jax.experimental.pallas import tpu_sc as plsc

D_MODEL = 768
D_FF = 3072
E = 8
TEMP = 0.7
N = 2048
NA = 2 * N

TM = 256
NB = NA // TM
TPAD = 24

NC, NS = 2, 16
NW = NC * NS
SPW = N // NW
CPW = N // NW


def _router_body(x_ref, wr_ref, br_ref,
                 wk_ref, posi_ref, cnts_ref, aux_ref):
    x = x_ref[...]
    logits = jnp.dot(x, wr_ref[...], preferred_element_type=jnp.float32)
    logits = logits + br_ref[...]
    z = logits / TEMP
    z = z - jnp.max(z, axis=1, keepdims=True)
    ez = jnp.exp(z)
    probs = ez / jnp.sum(ez, axis=1, keepdims=True)

    e_ids = lax.broadcasted_iota(jnp.int32, (N, E), 1)
    m1 = jnp.max(probs, axis=1, keepdims=True)
    i1 = jnp.min(jnp.where(probs == m1, e_ids, E), axis=1, keepdims=True)
    oh1 = (e_ids == i1).astype(jnp.float32)
    p2 = jnp.where(oh1 > 0, -jnp.inf, probs)
    m2 = jnp.max(p2, axis=1, keepdims=True)
    i2 = jnp.min(jnp.where(p2 == m2, e_ids, E), axis=1, keepdims=True)
    oh2 = (e_ids == i2).astype(jnp.float32)

    denom = m1 + m2 + 1e-6
    w1n = m1 / denom
    w2n = m2 / denom

    cum = oh1 + oh2
    k = 1
    while k < N:
        cum = cum + jnp.concatenate(
            [jnp.zeros((k, E), jnp.float32), cum[:N - k]], axis=0)
        k *= 2

    counts = cum[N - 1:N, :]
    offs = jnp.zeros_like(counts)
    for k in range(1, E):
        offs = offs + jnp.concatenate(
            [jnp.zeros((1, k), jnp.float32), counts[:, :E - k]], axis=1)

    slot = offs + cum - 1.0
    pos0 = jnp.sum(oh1 * slot, axis=1, keepdims=True)
    pos1 = jnp.sum(oh2 * slot, axis=1, keepdims=True)

    wk_ref[...] = jnp.concatenate([w1n, w2n], axis=1)
    posi_ref[...] = jnp.concatenate([pos0, pos1], axis=1).astype(jnp.int32)
    cnts_ref[...] = jnp.concatenate([counts, offs], axis=0).astype(jnp.int32)

    f_i = counts / N
    p_i = jnp.mean(probs, axis=0, keepdims=True)
    aux_ref[...] = E * jnp.sum(f_i * p_i, keepdims=True).reshape(1, 1)


def _router(x_flat, Wr, br):
    return pl.pallas_call(
        _router_body,
        out_shape=[
            jax.ShapeDtypeStruct((N, 2), jnp.float32),
            jax.ShapeDtypeStruct((N, 2), jnp.int32),
            jax.ShapeDtypeStruct((2, E), jnp.int32),
            jax.ShapeDtypeStruct((1, 1), jnp.float32),
        ],
    )(x_flat, Wr, br.reshape(1, E))


_INV_B = 512


def _inv_body(posi_ref, wk_ref, tok_ref, ws_ref):
    b = pl.program_id(0)
    slot = lax.broadcasted_iota(jnp.int32, (1, _INV_B), 1) + _INV_B * b
    m0 = (posi_ref[:, 0:1] == slot).astype(jnp.float32)
    m1 = (posi_ref[:, 1:2] == slot).astype(jnp.float32)
    t_ids = lax.broadcasted_iota(jnp.int32, (N, 1), 0).astype(jnp.float32)
    tok = jnp.sum((m0 + m1) * t_ids, axis=0, keepdims=True)
    w = jnp.sum(m0 * wk_ref[:, 0:1] + m1 * wk_ref[:, 1:2],
                axis=0, keepdims=True)
    tok_ref[0] = tok.astype(jnp.int32)
    ws_ref[0] = w


def _invert_w(posi, wk):
    nblk = NA // _INV_B
    return pl.pallas_call(
        _inv_body,
        grid=(nblk,),
        in_specs=[
            pl.BlockSpec((N, 2), lambda b: (0, 0)),
            pl.BlockSpec((N, 2), lambda b: (0, 0)),
        ],
        out_specs=[
            pl.BlockSpec((1, 1, _INV_B), lambda b: (b, 0, 0)),
            pl.BlockSpec((1, 1, _INV_B), lambda b: (b, 0, 0)),
        ],
        out_shape=[
            jax.ShapeDtypeStruct((nblk, 1, _INV_B), jnp.int32),
            jax.ShapeDtypeStruct((nblk, 1, _INV_B), jnp.float32),
        ],
    )(posi, wk)


@functools.lru_cache(maxsize=None)
def _sc_mesh():
    return plsc.VectorSubcoreMesh(
        core_axis_name="c", subcore_axis_name="s",
        num_cores=NC, num_subcores=NS)


GPW = NA // NW


def _sc_gather(tok_sorted, x_flat):
    @functools.partial(
        pl.kernel,
        out_type=jax.ShapeDtypeStruct((NA, D_MODEL), jnp.float32),
        mesh=_sc_mesh(),
        scratch_types=[
            pltpu.VMEM((GPW,), jnp.int32),
            pltpu.VMEM((GPW, D_MODEL), jnp.float32),
            pltpu.SemaphoreType.DMA,
        ],
    )
    def body(tok_hbm, x_hbm, out_hbm, idx_v, rows_v, sem):
        wid = lax.axis_index("s") * NC + lax.axis_index("c")
        base = wid * GPW
        pltpu.sync_copy(tok_hbm.at[pl.ds(base, GPW)], idx_v)
        pltpu.async_copy(x_hbm.at[idx_v], rows_v, sem).wait()
        pltpu.sync_copy(rows_v, out_hbm.at[pl.ds(base, GPW)])

    return body(tok_sorted, x_flat)


def _ffn_body(meta_ref, x_ref, w1_ref, b1_ref, w2_ref, b2_ref, ws_ref,
              out_ref):
    t = pl.program_id(0)
    r = meta_ref[1, t]
    r_prev = meta_ref[1, jnp.maximum(t - 1, 0)]
    first = (t == 0) | (r != r_prev)
    lo = meta_ref[2, t]
    hi = meta_ref[3, t]

    rows = lax.broadcasted_iota(jnp.int32, (TM, 1), 0)
    valid = (rows >= lo) & (rows < hi)

    x = x_ref[...]
    h = jnp.dot(x, w1_ref[0], preferred_element_type=jnp.float32)
    h = h + b1_ref[0]
    h = h * (1.0 / (1.0 + jnp.exp(-h)))
    contrib = jnp.dot(h, w2_ref[0], preferred_element_type=jnp.float32)
    contrib = contrib + b2_ref[0]
    update = jnp.where(valid, ws_ref[...] * contrib, 0.0)

    @pl.when(first)
    def _():
        out_ref[...] = update

    @pl.when(jnp.logical_not(first))
    def _():
        out_ref[...] += update


def _ffn_grouped(meta, x_sorted, W1, b1, W2, b2, w_sorted):
    grid_spec = pltpu.PrefetchScalarGridSpec(
        num_scalar_prefetch=1,
        grid=(TPAD,),
        in_specs=[
            pl.BlockSpec((TM, D_MODEL), lambda t, m: (m[1, t], 0)),
            pl.BlockSpec((1, D_MODEL, D_FF), lambda t, m: (m[0, t], 0, 0)),
            pl.BlockSpec((1, 1, D_FF), lambda t, m: (m[0, t], 0, 0)),
            pl.BlockSpec((1, D_FF, D_MODEL), lambda t, m: (m[0, t], 0, 0)),
            pl.BlockSpec((1, 1, D_MODEL), lambda t, m: (m[0, t], 0, 0)),
            pl.BlockSpec((TM, 1), lambda t, m: (m[1, t], 0)),
        ],
        out_specs=pl.BlockSpec((TM, D_MODEL), lambda t, m: (m[1, t], 0)),
    )
    return pl.pallas_call(
        _ffn_body,
        grid_spec=grid_spec,
        out_shape=jax.ShapeDtypeStruct((NA, D_MODEL), jnp.float32),
    )(meta, x_sorted, W1, b1.reshape(E, 1, D_FF), W2,
      b2.reshape(E, 1, D_MODEL), w_sorted)


def _sc_combine(i0, i1, out_sorted):
    @functools.partial(
        pl.kernel,
        out_type=jax.ShapeDtypeStruct((N, D_MODEL), jnp.float32),
        mesh=_sc_mesh(),
        scratch_types=[
            pltpu.VMEM((CPW,), jnp.int32),
            pltpu.VMEM((CPW,), jnp.int32),
            pltpu.VMEM((CPW, D_MODEL), jnp.float32),
            pltpu.VMEM((CPW, D_MODEL), jnp.float32),
            pltpu.SemaphoreType.DMA,
        ],
    )
    def body(i0_hbm, i1_hbm, os_hbm, out_hbm, idx0, idx1, r0, r1, sem):
        wid = lax.axis_index("s") * NC + lax.axis_index("c")
        base = wid * CPW
        pltpu.sync_copy(i0_hbm.at[pl.ds(base, CPW)], idx0)
        pltpu.sync_copy(i1_hbm.at[pl.ds(base, CPW)], idx1)
        pltpu.async_copy(os_hbm.at[idx0], r0, sem).wait()
        pltpu.async_copy(os_hbm.at[idx1], r1, sem).wait()

        def add_row(i, _):
            for j in range(D_MODEL // 16):
                sl = pl.ds(j * 16, 16)
                r0[i, sl] = r0[i, sl] + r1[i, sl]
            return 0

        lax.fori_loop(0, CPW, add_row, 0)
        pltpu.sync_copy(r0, out_hbm.at[pl.ds(base, CPW)])

    return body(i0, i1, out_sorted)


def _tile_meta(counts, offs):
    starts = offs
    ends = offs + counts
    r_ids = jnp.arange(NB, dtype=jnp.int32)[:, None]
    flag = ((starts[None, :] < (r_ids + 1) * TM)
            & (ends[None, :] > r_ids * TM)
            & (counts[None, :] > 0))
    tidx = jnp.cumsum(flag.reshape(-1).astype(jnp.int32)) - 1
    tgt = jnp.where(flag.reshape(-1), tidx, TPAD)
    lo = jnp.clip(jnp.broadcast_to(starts[None, :], (NB, E)) - r_ids * TM,
                  0, TM).reshape(-1)
    hi = jnp.clip(jnp.broadcast_to(ends[None, :], (NB, E)) - r_ids * TM,
                  0, TM).reshape(-1)
    e_flat = jnp.broadcast_to(jnp.arange(E, dtype=jnp.int32)[None, :],
                              (NB, E)).reshape(-1)
    r_flat = jnp.broadcast_to(r_ids, (NB, E)).reshape(-1)

    e_t = jnp.zeros((TPAD,), jnp.int32).at[tgt].set(e_flat, mode="drop")
    r_t = jnp.full((TPAD,), NB - 1, jnp.int32).at[tgt].set(r_flat,
                                                           mode="drop")
    lo_t = jnp.zeros((TPAD,), jnp.int32).at[tgt].set(lo, mode="drop")
    hi_t = jnp.zeros((TPAD,), jnp.int32).at[tgt].set(hi, mode="drop")
    return jnp.stack([e_t, r_t, lo_t, hi_t])


@jax.jit
def kernel(x, Wr, br, W1, b1, W2, b2):
    B, L, D = x.shape
    x_flat = x.reshape(-1, D)
    wk, posi, cnts, aux = _router(x_flat, Wr, br)
    tok3, ws3 = _invert_w(posi, wk)
    tok_sorted = tok3.reshape(-1)
    w_sorted = ws3.reshape(-1, 1)
    meta = _tile_meta(cnts[0], cnts[1])
    i0 = posi[:, 0]
    i1 = posi[:, 1]
    x_sorted = _sc_gather(tok_sorted, x_flat)
    out_sorted = _ffn_grouped(meta, x_sorted, W1, b1, W2, b2, w_sorted)
    out = _sc_combine(i0, i1, out_sorted)
    return out.reshape(B, L, D), aux.reshape(1)
